# Initial kernel scaffold; baseline (speedup 1.0000x reference)
import jax
import jax.numpy as jnp
from jax.experimental import pallas as pl


def kernel(x, adj, sub_fea, W1, b1, W2, b2, fc1_W, fc1_b, fus_W, fus_b):
    raise NotImplementedError("write your pallas kernel here")


# speedup vs baseline: 1.0799x; 1.0799x over previous
import functools

import jax
import jax.numpy as jnp
from jax.experimental import pallas as pl
from jax.experimental.pallas import tpu as pltpu

_BF = jnp.bfloat16
_F32 = jnp.float32

_SELU_ALPHA = 1.6732632423543772848170429916717
_SELU_SCALE = 1.0507009873554804934193349852946


def _support1_body(x_ref, w1_ref, o_ref):
    o_ref[...] = jnp.dot(
        x_ref[...].astype(_BF), w1_ref[...], preferred_element_type=_F32
    ).astype(_BF)


def _layer1_body(adj_ref, s1_ref, w2_ref, b1_ref, o_ref):
    a = adj_ref[...].astype(_BF)
    acc = jnp.dot(a, s1_ref[...], preferred_element_type=_F32)
    h = jnp.maximum(acc + b1_ref[...], 0.0).astype(_BF)
    o_ref[...] = jnp.dot(h, w2_ref[...], preferred_element_type=_F32).astype(_BF)


def _layer2_body(
    adj_ref, s2_ref, b2_ref, sub_ref, fc1wt_ref, fc1b_ref, fuswt_ref,
    fusw_ref, fusb_ref, out_ref, l1_ref, gacc_ref, *, nm, bm, n_rows
):
    i = pl.program_id(0)
    a = adj_ref[...].astype(_BF)
    acc = jnp.dot(a, s2_ref[...], preferred_element_type=_F32)
    h2 = jnp.maximum(acc + b2_ref[...], 0.0)
    row = jax.lax.broadcasted_iota(jnp.int32, h2.shape, 0)
    h2 = jnp.where(row < (n_rows - i * bm), h2, 0.0)
    rs = jnp.sum(h2, axis=0, keepdims=True)

    @pl.when(i == 0)
    def _():
        gacc_ref[...] = rs

    @pl.when(i > 0)
    def _():
        gacc_ref[...] = gacc_ref[...] + rs

    @pl.when(i == nm - 1)
    def _():
        nclass = gacc_ref.shape[1]
        mean_h2 = gacc_ref[...] / jnp.float32(n_rows)
        g = _SELU_SCALE * jnp.where(
            mean_h2 > 0, mean_h2, _SELU_ALPHA * (jnp.exp(mean_h2) - 1.0)
        )
        x_ext = (
            jnp.dot(sub_ref[...].astype(_BF), fc1wt_ref[...],
                    preferred_element_type=_F32)
            + fc1b_ref[...]
        )
        out = (
            jnp.dot(g.astype(_BF), fuswt_ref[pl.ds(0, nclass), :],
                    preferred_element_type=_F32)
            + jnp.dot(x_ext.astype(_BF), fuswt_ref[pl.ds(nclass, nclass), :],
                      preferred_element_type=_F32)
            + fusb_ref[...]
        )
        m = jnp.max(out, axis=1, keepdims=True)
        e = out - m
        lse = jnp.log(jnp.sum(jnp.exp(e), axis=1, keepdims=True))
        out_ref[...] = e - lse
        l1_ref[...] = jnp.mean(jnp.abs(fusw_ref[...])).reshape(1, 1)


@jax.jit
def kernel(x, adj, sub_fea, W1, b1, W2, b2, fc1_W, fc1_b, fus_W, fus_b):
    n, nfeat = x.shape
    nhid = W1.shape[1]
    nclass = W2.shape[1]

    w1b = W1.astype(_BF)
    w2b = W2.astype(_BF)
    fc1wt = fc1_W.T.astype(_BF)
    fuswt = fus_W.T.astype(_BF)
    b1r = b1.reshape(1, nhid)
    b2r = b2.reshape(1, nclass)
    fc1br = fc1_b.reshape(1, nclass)
    fusbr = fus_b.reshape(1, nclass)

    bm0 = 1000 if n % 1000 == 0 else n
    s1 = pl.pallas_call(
        _support1_body,
        grid=(n // bm0,),
        in_specs=[
            pl.BlockSpec((bm0, nfeat), lambda i: (i, 0)),
            pl.BlockSpec((nfeat, nhid), lambda i: (0, 0)),
        ],
        out_specs=pl.BlockSpec((bm0, nhid), lambda i: (i, 0)),
        out_shape=jax.ShapeDtypeStruct((n, nhid), _BF),
    )(x, w1b)

    bm = 256 if n > 256 else n
    nm = (n + bm - 1) // bm

    s2 = pl.pallas_call(
        _layer1_body,
        grid=(nm,),
        in_specs=[
            pl.BlockSpec((bm, n), lambda i: (i, 0)),
            pl.BlockSpec((n, nhid), lambda i: (0, 0)),
            pl.BlockSpec((nhid, nclass), lambda i: (0, 0)),
            pl.BlockSpec((1, nhid), lambda i: (0, 0)),
        ],
        out_specs=pl.BlockSpec((bm, nclass), lambda i: (i, 0)),
        out_shape=jax.ShapeDtypeStruct((n, nclass), _BF),
        compiler_params=pltpu.CompilerParams(
            dimension_semantics=("arbitrary",),
        ),
    )(adj, s1, w2b, b1r)

    logp, l1 = pl.pallas_call(
        functools.partial(_layer2_body, nm=nm, bm=bm, n_rows=n),
        grid=(nm,),
        in_specs=[
            pl.BlockSpec((bm, n), lambda i: (i, 0)),
            pl.BlockSpec((n, nclass), lambda i: (0, 0)),
            pl.BlockSpec((1, nclass), lambda i: (0, 0)),
            pl.BlockSpec(sub_fea.shape, lambda i: (0, 0)),
            pl.BlockSpec(fc1wt.shape, lambda i: (0, 0)),
            pl.BlockSpec((1, nclass), lambda i: (0, 0)),
            pl.BlockSpec(fuswt.shape, lambda i: (0, 0)),
            pl.BlockSpec(fus_W.shape, lambda i: (0, 0)),
            pl.BlockSpec((1, nclass), lambda i: (0, 0)),
        ],
        out_specs=[
            pl.BlockSpec((1, nclass), lambda i: (0, 0)),
            pl.BlockSpec((1, 1), lambda i: (0, 0)),
        ],
        out_shape=[
            jax.ShapeDtypeStruct((1, nclass), _F32),
            jax.ShapeDtypeStruct((1, 1), _F32),
        ],
        scratch_shapes=[
            pltpu.VMEM((1, nclass), _F32),
        ],
        compiler_params=pltpu.CompilerParams(
            dimension_semantics=("arbitrary",),
        ),
    )(adj, s2, b2r, sub_fea, fc1wt, fc1br, fuswt, fus_W, fusbr)

    return logp, l1.reshape(())


# speedup vs baseline: 1.1158x; 1.0332x over previous
import functools

import jax
import jax.numpy as jnp
from jax.experimental import pallas as pl
from jax.experimental.pallas import tpu as pltpu

_BF = jnp.bfloat16
_F32 = jnp.float32

_SELU_ALPHA = 1.6732632423543772848170429916717
_SELU_SCALE = 1.0507009873554804934193349852946


def _mega_body(
    x_ref, adj_ref, w1_ref, w2_ref, b1_ref, b2_ref, sub_ref, fc1wt_ref,
    fc1b_ref, fuswt_ref, fusw_ref, fusb_ref, out_ref, l1_ref,
    s1_scr, s2_scr, gacc_ref, *, p0, nm, bm, bm0, n_rows
):
    t = pl.program_id(0)

    @pl.when(t < p0)
    def _():
        blk = jnp.dot(
            x_ref[...].astype(_BF), w1_ref[...], preferred_element_type=_F32
        ).astype(_BF)
        s1_scr[pl.ds(t * bm0, bm0), :] = blk

    @pl.when((t >= p0) & (t < p0 + nm))
    def _():
        i = t - p0
        a = adj_ref[...].astype(_BF)
        acc = jnp.dot(a, s1_scr[...], preferred_element_type=_F32)
        h = jnp.maximum(acc + b1_ref[...], 0.0).astype(_BF)
        s2_scr[pl.ds(i * bm, bm), :] = jnp.dot(
            h, w2_ref[...], preferred_element_type=_F32
        ).astype(_BF)

    @pl.when(t >= p0 + nm)
    def _():
        i = t - p0 - nm
        a = adj_ref[...].astype(_BF)
        acc = jnp.dot(
            a, s2_scr[: s1_scr.shape[0], :], preferred_element_type=_F32
        )
        h2 = jnp.maximum(acc + b2_ref[...], 0.0)
        row = jax.lax.broadcasted_iota(jnp.int32, h2.shape, 0)
        h2 = jnp.where(row < (n_rows - i * bm), h2, 0.0)
        rs = jnp.sum(h2, axis=0, keepdims=True)

        @pl.when(i == 0)
        def _():
            gacc_ref[...] = rs

        @pl.when(i > 0)
        def _():
            gacc_ref[...] = gacc_ref[...] + rs

        @pl.when(i == nm - 1)
        def _():
            nclass = gacc_ref.shape[1]
            mean_h2 = gacc_ref[...] / jnp.float32(n_rows)
            g = _SELU_SCALE * jnp.where(
                mean_h2 > 0, mean_h2, _SELU_ALPHA * (jnp.exp(mean_h2) - 1.0)
            )
            x_ext = (
                jnp.dot(sub_ref[...].astype(_BF), fc1wt_ref[...],
                        preferred_element_type=_F32)
                + fc1b_ref[...]
            )
            out = (
                jnp.dot(g.astype(_BF), fuswt_ref[pl.ds(0, nclass), :],
                        preferred_element_type=_F32)
                + jnp.dot(x_ext.astype(_BF), fuswt_ref[pl.ds(nclass, nclass), :],
                          preferred_element_type=_F32)
                + fusb_ref[...]
            )
            m = jnp.max(out, axis=1, keepdims=True)
            e = out - m
            lse = jnp.log(jnp.sum(jnp.exp(e), axis=1, keepdims=True))
            out_ref[...] = e - lse
            l1_ref[...] = jnp.mean(jnp.abs(fusw_ref[...])).reshape(1, 1)


@jax.jit
def kernel(x, adj, sub_fea, W1, b1, W2, b2, fc1_W, fc1_b, fus_W, fus_b):
    n, nfeat = x.shape
    nhid = W1.shape[1]
    nclass = W2.shape[1]

    w1b = W1.astype(_BF)
    w2b = W2.astype(_BF)
    fc1wt = fc1_W.T.astype(_BF)
    fuswt = fus_W.T.astype(_BF)
    b1r = b1.reshape(1, nhid)
    b2r = b2.reshape(1, nclass)
    fc1br = fc1_b.reshape(1, nclass)
    fusbr = fus_b.reshape(1, nclass)

    bm0 = 1000 if n % 1000 == 0 else n
    p0 = n // bm0
    bm = 256 if n > 256 else n
    nm = (n + bm - 1) // bm
    npad = nm * bm
    grid = (p0 + 2 * nm,)

    def x_map(t):
        return (jnp.minimum(t, p0 - 1), 0)

    def adj_map(t):
        i1 = jnp.maximum(t - p0, 0)
        i2 = jnp.maximum(t - p0 - nm, 0)
        return (jnp.where(t < p0 + nm, i1, i2), 0)

    logp, l1 = pl.pallas_call(
        functools.partial(
            _mega_body, p0=p0, nm=nm, bm=bm, bm0=bm0, n_rows=n
        ),
        grid=grid,
        in_specs=[
            pl.BlockSpec((bm0, nfeat), x_map),
            pl.BlockSpec((bm, n), adj_map),
            pl.BlockSpec((nfeat, nhid), lambda t: (0, 0)),
            pl.BlockSpec((nhid, nclass), lambda t: (0, 0)),
            pl.BlockSpec((1, nhid), lambda t: (0, 0)),
            pl.BlockSpec((1, nclass), lambda t: (0, 0)),
            pl.BlockSpec(sub_fea.shape, lambda t: (0, 0)),
            pl.BlockSpec(fc1wt.shape, lambda t: (0, 0)),
            pl.BlockSpec((1, nclass), lambda t: (0, 0)),
            pl.BlockSpec(fuswt.shape, lambda t: (0, 0)),
            pl.BlockSpec(fus_W.shape, lambda t: (0, 0)),
            pl.BlockSpec((1, nclass), lambda t: (0, 0)),
        ],
        out_specs=[
            pl.BlockSpec((1, nclass), lambda t: (0, 0)),
            pl.BlockSpec((1, 1), lambda t: (0, 0)),
        ],
        out_shape=[
            jax.ShapeDtypeStruct((1, nclass), _F32),
            jax.ShapeDtypeStruct((1, 1), _F32),
        ],
        scratch_shapes=[
            pltpu.VMEM((n, nhid), _BF),
            pltpu.VMEM((npad, nclass), _BF),
            pltpu.VMEM((1, nclass), _F32),
        ],
        compiler_params=pltpu.CompilerParams(
            dimension_semantics=("arbitrary",),
        ),
    )(x, adj, w1b, w2b, b1r, b2r, sub_fea, fc1wt, fc1br, fuswt, fus_W, fusbr)

    return logp, l1.reshape(())


# speedup vs baseline: 1.1830x; 1.0602x over previous
import functools

import jax
import jax.numpy as jnp
from jax.experimental import pallas as pl
from jax.experimental.pallas import tpu as pltpu

_BF = jnp.bfloat16
_F32 = jnp.float32

_SELU_ALPHA = 1.6732632423543772848170429916717
_SELU_SCALE = 1.0507009873554804934193349852946


def _mega_body(
    x_ref, adj_ref, w1_ref, w2_ref, b1_ref, b2_ref, sub_ref, fc1wt_ref,
    fc1b_ref, fuswt_ref, fusw_ref, fusb_ref, out_ref, l1_ref,
    s1_scr, s2_scr, gacc_ref, *, p0, nm, bm, bm0, n_rows
):
    t = pl.program_id(0)

    @pl.when(t < p0)
    def _():
        blk = jnp.dot(
            x_ref[...].astype(_BF), w1_ref[...], preferred_element_type=_F32
        ).astype(_BF)
        s1_scr[pl.ds(t * bm0, bm0), :] = blk

    @pl.when((t >= p0) & (t < p0 + nm))
    def _():
        i = t - p0
        a = adj_ref[...].astype(_BF)
        acc = jnp.dot(a, s1_scr[...], preferred_element_type=_F32)
        h = jnp.maximum(acc + b1_ref[...], 0.0).astype(_BF)
        s2_scr[pl.ds(i * bm, bm), :] = jnp.dot(
            h, w2_ref[...], preferred_element_type=_F32
        ).astype(_BF)

    @pl.when(t >= p0 + nm)
    def _():
        i = t - p0 - nm
        a = adj_ref[...].astype(_BF)
        acc = jnp.dot(
            a, s2_scr[: s1_scr.shape[0], :], preferred_element_type=_F32
        )
        h2 = jnp.maximum(acc + b2_ref[...], 0.0)
        row = jax.lax.broadcasted_iota(jnp.int32, h2.shape, 0)
        h2 = jnp.where(row < (n_rows - i * bm), h2, 0.0)
        rs = jnp.sum(h2, axis=0, keepdims=True)

        @pl.when(i == 0)
        def _():
            gacc_ref[...] = rs

        @pl.when(i > 0)
        def _():
            gacc_ref[...] = gacc_ref[...] + rs

        @pl.when(i == nm - 1)
        def _():
            nclass = gacc_ref.shape[1]
            mean_h2 = gacc_ref[...] / jnp.float32(n_rows)
            g = _SELU_SCALE * jnp.where(
                mean_h2 > 0, mean_h2, _SELU_ALPHA * (jnp.exp(mean_h2) - 1.0)
            )
            x_ext = (
                jnp.dot(sub_ref[...].astype(_BF), fc1wt_ref[...],
                        preferred_element_type=_F32)
                + fc1b_ref[...]
            )
            out = (
                jnp.dot(g.astype(_BF), fuswt_ref[pl.ds(0, nclass), :],
                        preferred_element_type=_F32)
                + jnp.dot(x_ext.astype(_BF), fuswt_ref[pl.ds(nclass, nclass), :],
                          preferred_element_type=_F32)
                + fusb_ref[...]
            )
            m = jnp.max(out, axis=1, keepdims=True)
            e = out - m
            lse = jnp.log(jnp.sum(jnp.exp(e), axis=1, keepdims=True))
            out_ref[...] = e - lse
            l1_ref[...] = jnp.mean(jnp.abs(fusw_ref[...])).reshape(1, 1)


@jax.jit
def kernel(x, adj, sub_fea, W1, b1, W2, b2, fc1_W, fc1_b, fus_W, fus_b):
    n, nfeat = x.shape
    nhid = W1.shape[1]
    nclass = W2.shape[1]

    w1b = W1.astype(_BF)
    w2b = W2.astype(_BF)
    fc1wt = fc1_W.T.astype(_BF)
    fuswt = fus_W.T.astype(_BF)
    b1r = b1.reshape(1, nhid)
    b2r = b2.reshape(1, nclass)
    fc1br = fc1_b.reshape(1, nclass)
    fusbr = fus_b.reshape(1, nclass)

    bm0 = 2000 if n % 2000 == 0 else n
    p0 = n // bm0
    bm = 400 if n > 400 else n
    nm = (n + bm - 1) // bm
    npad = nm * bm
    grid = (p0 + 2 * nm,)

    def x_map(t):
        return (jnp.minimum(t, p0 - 1), 0)

    def adj_map(t):
        i1 = jnp.maximum(t - p0, 0)
        i2 = jnp.maximum(t - p0 - nm, 0)
        return (jnp.where(t < p0 + nm, i1, i2), 0)

    logp, l1 = pl.pallas_call(
        functools.partial(
            _mega_body, p0=p0, nm=nm, bm=bm, bm0=bm0, n_rows=n
        ),
        grid=grid,
        in_specs=[
            pl.BlockSpec((bm0, nfeat), x_map),
            pl.BlockSpec((bm, n), adj_map),
            pl.BlockSpec((nfeat, nhid), lambda t: (0, 0)),
            pl.BlockSpec((nhid, nclass), lambda t: (0, 0)),
            pl.BlockSpec((1, nhid), lambda t: (0, 0)),
            pl.BlockSpec((1, nclass), lambda t: (0, 0)),
            pl.BlockSpec(sub_fea.shape, lambda t: (0, 0)),
            pl.BlockSpec(fc1wt.shape, lambda t: (0, 0)),
            pl.BlockSpec((1, nclass), lambda t: (0, 0)),
            pl.BlockSpec(fuswt.shape, lambda t: (0, 0)),
            pl.BlockSpec(fus_W.shape, lambda t: (0, 0)),
            pl.BlockSpec((1, nclass), lambda t: (0, 0)),
        ],
        out_specs=[
            pl.BlockSpec((1, nclass), lambda t: (0, 0)),
            pl.BlockSpec((1, 1), lambda t: (0, 0)),
        ],
        out_shape=[
            jax.ShapeDtypeStruct((1, nclass), _F32),
            jax.ShapeDtypeStruct((1, 1), _F32),
        ],
        scratch_shapes=[
            pltpu.VMEM((n, nhid), _BF),
            pltpu.VMEM((npad, nclass), _BF),
            pltpu.VMEM((1, nclass), _F32),
        ],
        compiler_params=pltpu.CompilerParams(
            dimension_semantics=("arbitrary",),
        ),
    )(x, adj, w1b, w2b, b1r, b2r, sub_fea, fc1wt, fc1br, fuswt, fus_W, fusbr)

    return logp, l1.reshape(())
